# baseline (speedup 1.0000x reference)
import jax
import jax.numpy as jnp
from jax.experimental import pallas as pl
from jax.experimental.pallas import tpu as pltpu

_R = 288
_C = 128
_G = 64
_T = _R // 8


def _smooth_l1(d):
    ad = jnp.abs(d)
    return jnp.where(ad < 1.0, 0.5 * d * d, ad - 0.5)


def _loss_kernel(hw_ref, gt_ref, gtv_ref, at_ref, st_ref, dt_ref,
                 cls_ref, reg_ref, amax_s, aidx_s, cmax_s, cidx_s):
    h = hw_ref[0, 0]
    w = hw_ref[0, 1]
    sub = jax.lax.broadcasted_iota(jnp.int32, (8, _C), 0)
    lane = jax.lax.broadcasted_iota(jnp.int32, (8, _C), 1)
    base_idx = sub * _C + lane

    cmax_s[...] = jnp.full((_G, 8, _C), -1.0, jnp.float32)
    cidx_s[...] = jnp.zeros((_G, 8, _C), jnp.int32)

    def tile_body(t, _):
        ax1 = at_ref[0, pl.ds(8 * t, 8), :]
        ay1 = at_ref[1, pl.ds(8 * t, 8), :]
        ax2 = at_ref[2, pl.ds(8 * t, 8), :]
        ay2 = at_ref[3, pl.ds(8 * t, 8), :]
        validt = (ax1 >= 0.0) & (ay1 >= 0.0) & (ax2 <= w) & (ay2 <= h)
        ax1d = jnp.where(validt, ax1, 2048.0)
        ax2d = jnp.where(validt, ax2, 2048.0)
        area_a = (ax2d - ax1d) * (ay2 - ay1)
        idxt = base_idx + t * (8 * _C)

        def g_body(i, carry):
            amax, aidx = carry
            for j in range(4):
                g = 4 * i + j
                gx1 = gt_ref[0, g]
                gy1 = gt_ref[1, g]
                gx2 = gt_ref[2, g]
                gy2 = gt_ref[3, g]
                garea = (gx2 - gx1) * (gy2 - gy1)
                ix1 = jnp.maximum(ax1d, gx1)
                iy1 = jnp.maximum(ay1, gy1)
                ix2 = jnp.minimum(ax2d, gx2)
                iy2 = jnp.minimum(ay2, gy2)
                iw = jnp.maximum(ix2 - ix1, 0.0)
                ih = iy2 - iy1
                inter = iw * ih
                union = (area_a + garea) - inter
                iou = inter / union
                upd = iou > amax
                amax = jnp.maximum(amax, iou)
                aidx = jnp.where(upd, g, aidx)
                cm = cmax_s[g]
                updc = iou > cm
                cmax_s[g] = jnp.maximum(cm, iou)
                cidx_s[g] = jnp.where(updc, idxt, cidx_s[g])
            return amax, aidx

        amax, aidx = jax.lax.fori_loop(
            0, _G // 4, g_body,
            (jnp.full((8, _C), -1.0, jnp.float32),
             jnp.zeros((8, _C), jnp.int32)))
        amax_s[pl.ds(8 * t, 8), :] = amax
        aidx_s[pl.ds(8 * t, 8), :] = aidx
        return 0

    jax.lax.fori_loop(0, _T, tile_body, 0)

    ridx = jax.lax.broadcasted_iota(jnp.int32, (_R, _C), 0)
    cidx_full = jax.lax.broadcasted_iota(jnp.int32, (_R, _C), 1)
    idx_full = ridx * _C + cidx_full
    forced = jnp.zeros((_R, _C), jnp.bool_)
    for g in range(_G):
        maxv = jnp.max(cmax_s[g])
        cand = jnp.where(cmax_s[g] == maxv, cidx_s[g], jnp.int32(2**30))
        gidx = jnp.min(cand)
        forced = forced | (idx_full == gidx)

    ax1 = at_ref[0]
    ay1 = at_ref[1]
    ax2 = at_ref[2]
    ay2 = at_ref[3]
    valid = (ax1 >= 0.0) & (ay1 >= 0.0) & (ax2 <= w) & (ay2 <= h)
    amax = amax_s[...]
    pos = valid & (forced | (amax >= 0.7))
    neg = valid & (amax < 0.3) & jnp.logical_not(forced)

    ic0 = jax.lax.broadcasted_iota(jnp.int32, (_C, _C), 0)
    ic1 = jax.lax.broadcasted_iota(jnp.int32, (_C, _C), 1)
    u_in = (ic0 <= ic1).astype(jnp.float32)
    ir0 = jax.lax.broadcasted_iota(jnp.int32, (_R, _R), 0)
    ir1 = jax.lax.broadcasted_iota(jnp.int32, (_R, _R), 1)
    t_ex = (ir1 < ir0).astype(jnp.float32)

    pos_f = pos.astype(jnp.float32)
    neg_f = neg.astype(jnp.float32)
    pos_rs = jnp.sum(pos_f, axis=1, keepdims=True)
    neg_rs = jnp.sum(neg_f, axis=1, keepdims=True)
    pc = (jnp.dot(t_ex, pos_rs, preferred_element_type=jnp.float32)
          + jnp.dot(pos_f, u_in, preferred_element_type=jnp.float32))
    nc = (jnp.dot(t_ex, neg_rs, preferred_element_type=jnp.float32)
          + jnp.dot(neg_f, u_in, preferred_element_type=jnp.float32))

    npos_total = jnp.sum(pos_f)
    npos = jnp.minimum(npos_total, 128.0)
    sel_pos = pos & (pc <= 128.0)
    nneg_total = jnp.sum(neg_f)
    nneg = jnp.minimum(nneg_total, 256.0 - npos)
    sel_neg = neg & (nc <= 256.0 - npos)

    s0 = st_ref[0]
    s1 = st_ref[1]
    mx = jnp.maximum(s0, s1)
    lse = mx + jnp.log(jnp.exp(s0 - mx) + jnp.exp(s1 - mx))
    ce = jnp.where(sel_pos, lse - s1, 0.0) + jnp.where(sel_neg, lse - s0, 0.0)
    cls_loss = 3.0 * jnp.sum(ce) / (npos + nneg)

    gvx1 = gtv_ref[0, 0:1, :]
    gvy1 = gtv_ref[0, 1:2, :]
    gvx2 = gtv_ref[0, 2:3, :]
    gvy2 = gtv_ref[0, 3:4, :]
    gw_row = gvx2 - gvx1
    gh_row = gvy2 - gvy1
    gcx_row = gvx1 + 0.5 * gw_row
    gcy_row = gvy1 + 0.5 * gh_row
    aidx = aidx_s[...]
    gcx = jnp.take_along_axis(jnp.broadcast_to(gcx_row, (_R, _G)), aidx, axis=1)
    gcy = jnp.take_along_axis(jnp.broadcast_to(gcy_row, (_R, _G)), aidx, axis=1)
    gwp = jnp.take_along_axis(jnp.broadcast_to(gw_row, (_R, _G)), aidx, axis=1)
    ghp = jnp.take_along_axis(jnp.broadcast_to(gh_row, (_R, _G)), aidx, axis=1)

    aw = ax2 - ax1
    ah = ay2 - ay1
    acx = ax1 + 0.5 * aw
    acy = ay1 + 0.5 * ah
    tx = (gcx - acx) / aw
    ty = (gcy - acy) / ah
    tw = jnp.log(gwp / aw)
    th = jnp.log(ghp / ah)
    rows = (_smooth_l1(dt_ref[0] - tx) + _smooth_l1(dt_ref[1] - ty)
            + _smooth_l1(dt_ref[2] - tw) + _smooth_l1(dt_ref[3] - th))
    reg_sum = jnp.sum(jnp.where(sel_pos, rows, 0.0))
    reg = reg_sum / jnp.maximum(npos, 1.0)
    reg = jnp.where(npos > 0.0, reg, 0.0)

    cls_ref[0, 0] = cls_loss
    reg_ref[0, 0] = reg


def kernel(image_shape, anchors, rpn_score, rpn_bboxes_txtytwth, gt_bboxes):
    hw = image_shape.astype(jnp.float32).reshape(1, 2)
    gtt = gt_bboxes.T.reshape(4, _G)
    gtv = gtt.reshape(1, 4, _G)
    at = anchors.T.reshape(4, _R, _C)
    st = rpn_score.T.reshape(2, _R, _C)
    dt = rpn_bboxes_txtytwth.T.reshape(4, _R, _C)
    cls_out, reg_out = pl.pallas_call(
        _loss_kernel,
        out_shape=[jax.ShapeDtypeStruct((1, 1), jnp.float32)] * 2,
        in_specs=[
            pl.BlockSpec(memory_space=pltpu.SMEM),
            pl.BlockSpec(memory_space=pltpu.SMEM),
            pl.BlockSpec(memory_space=pltpu.VMEM),
            pl.BlockSpec(memory_space=pltpu.VMEM),
            pl.BlockSpec(memory_space=pltpu.VMEM),
            pl.BlockSpec(memory_space=pltpu.VMEM),
        ],
        out_specs=[pl.BlockSpec(memory_space=pltpu.SMEM)] * 2,
        scratch_shapes=[pltpu.VMEM((_R, _C), jnp.float32),
                        pltpu.VMEM((_R, _C), jnp.int32),
                        pltpu.VMEM((_G, 8, _C), jnp.float32),
                        pltpu.VMEM((_G, 8, _C), jnp.int32)],
    )(hw, gtt, gtv, at, st, dt)
    return (cls_out.reshape(()), reg_out.reshape(()))


# speedup vs baseline: 1.8023x; 1.8023x over previous
import jax
import jax.numpy as jnp
from jax.experimental import pallas as pl
from jax.experimental.pallas import tpu as pltpu

_R = 288
_C = 128
_G = 64
_T = _R // 8


def _smooth_l1(d):
    ad = jnp.abs(d)
    return jnp.where(ad < 1.0, 0.5 * d * d, ad - 0.5)


def _loss_kernel(hw_ref, gt_ref, gtv_ref, all_ref,
                 cls_ref, reg_ref, amax_s, aidx_s):
    at_ref = all_ref.at[0:4]
    dt_ref = all_ref.at[4:8]
    st_ref = all_ref.at[8:10]
    h = hw_ref[0, 0]
    w = hw_ref[0, 1]
    gs = []
    for g in range(_G):
        gx1 = gt_ref[0, g]
        gy1 = gt_ref[1, g]
        gx2 = gt_ref[2, g]
        gy2 = gt_ref[3, g]
        gs.append((gx1, gy1, gx2, gy2, (gx2 - gx1) * (gy2 - gy1)))

    ts = 32
    nt = _R // ts
    sub = jax.lax.broadcasted_iota(jnp.int32, (ts, _C), 0)
    lane = jax.lax.broadcasted_iota(jnp.int32, (ts, _C), 1)
    base_idx = sub * _C + lane

    cmax = [jnp.full((ts, _C), -1.0, jnp.float32) for _ in range(_G)]
    cidx = [jnp.zeros((ts, _C), jnp.int32) for _ in range(_G)]

    for t in range(nt):
        sl = slice(ts * t, ts * t + ts)
        ax1 = at_ref[0, sl, :]
        ay1 = at_ref[1, sl, :]
        ax2 = at_ref[2, sl, :]
        ay2 = at_ref[3, sl, :]
        validt = (ax1 >= 0.0) & (ay1 >= 0.0) & (ax2 <= w) & (ay2 <= h)
        ax1d = jnp.where(validt, ax1, 2048.0)
        ax2d = jnp.where(validt, ax2, 2048.0)
        area_a = (ax2d - ax1d) * (ay2 - ay1)
        idxt = base_idx + t * (ts * _C)
        nch = 2
        amax_c = [jnp.full((ts, _C), -1.0, jnp.float32) for _ in range(nch)]
        aidx_c = [jnp.zeros((ts, _C), jnp.int32) for _ in range(nch)]
        for g in range(_G):
            k = g % nch
            gx1, gy1, gx2, gy2, garea = gs[g]
            ix1 = jnp.maximum(ax1d, gx1)
            iy1 = jnp.maximum(ay1, gy1)
            ix2 = jnp.minimum(ax2d, gx2)
            iy2 = jnp.minimum(ay2, gy2)
            iw = jnp.maximum(ix2 - ix1, 0.0)
            ih = jnp.maximum(iy2 - iy1, 0.0)
            inter = iw * ih
            union = (area_a + garea) - inter
            iou = inter / union
            upd = iou > amax_c[k]
            amax_c[k] = jnp.maximum(amax_c[k], iou)
            aidx_c[k] = jnp.where(upd, g, aidx_c[k])
            updc = iou > cmax[g]
            cmax[g] = jnp.maximum(cmax[g], iou)
            cidx[g] = jnp.where(updc, idxt, cidx[g])
        amax, aidx = amax_c[0], aidx_c[0]
        for k in range(1, nch):
            take = (amax_c[k] > amax) | ((amax_c[k] == amax)
                                         & (aidx_c[k] < aidx))
            amax = jnp.maximum(amax, amax_c[k])
            aidx = jnp.where(take, aidx_c[k], aidx)
        amax_s[sl, :] = amax
        aidx_s[sl, :] = aidx

    ridx = jax.lax.broadcasted_iota(jnp.int32, (_R, _C), 0)
    cidx_full = jax.lax.broadcasted_iota(jnp.int32, (_R, _C), 1)
    idx_full = ridx * _C + cidx_full
    forced = jnp.zeros((_R, _C), jnp.bool_)
    for g in range(_G):
        maxv = jnp.max(cmax[g])
        cand = jnp.where(cmax[g] == maxv, cidx[g], jnp.int32(2**30))
        gidx = jnp.min(cand)
        forced = forced | (idx_full == gidx)

    ax1 = at_ref[0]
    ay1 = at_ref[1]
    ax2 = at_ref[2]
    ay2 = at_ref[3]
    valid = (ax1 >= 0.0) & (ay1 >= 0.0) & (ax2 <= w) & (ay2 <= h)
    amax = amax_s[...]
    pos = valid & (forced | (amax >= 0.7))
    neg = valid & (amax < 0.3) & jnp.logical_not(forced)

    ic0 = jax.lax.broadcasted_iota(jnp.int32, (_C, _C), 0)
    ic1 = jax.lax.broadcasted_iota(jnp.int32, (_C, _C), 1)
    u_in = (ic0 <= ic1).astype(jnp.float32)
    ir0 = jax.lax.broadcasted_iota(jnp.int32, (_R, _R), 0)
    ir1 = jax.lax.broadcasted_iota(jnp.int32, (_R, _R), 1)
    t_ex = (ir1 < ir0).astype(jnp.float32)

    pos_f = pos.astype(jnp.float32)
    neg_f = neg.astype(jnp.float32)
    pos_rs = jnp.sum(pos_f, axis=1, keepdims=True)
    neg_rs = jnp.sum(neg_f, axis=1, keepdims=True)
    pc = (jnp.dot(t_ex, pos_rs, preferred_element_type=jnp.float32)
          + jnp.dot(pos_f, u_in, preferred_element_type=jnp.float32))
    nc = (jnp.dot(t_ex, neg_rs, preferred_element_type=jnp.float32)
          + jnp.dot(neg_f, u_in, preferred_element_type=jnp.float32))

    npos_total = jnp.sum(pos_f)
    npos = jnp.minimum(npos_total, 128.0)
    sel_pos = pos & (pc <= 128.0)
    nneg_total = jnp.sum(neg_f)
    nneg = jnp.minimum(nneg_total, 256.0 - npos)
    sel_neg = neg & (nc <= 256.0 - npos)

    s0 = st_ref[0]
    s1 = st_ref[1]
    mx = jnp.maximum(s0, s1)
    lse = mx + jnp.log(jnp.exp(s0 - mx) + jnp.exp(s1 - mx))
    ce = jnp.where(sel_pos, lse - s1, 0.0) + jnp.where(sel_neg, lse - s0, 0.0)
    cls_loss = 3.0 * jnp.sum(ce) / (npos + nneg)

    gvx1 = gtv_ref[0, 0:1, :]
    gvy1 = gtv_ref[0, 1:2, :]
    gvx2 = gtv_ref[0, 2:3, :]
    gvy2 = gtv_ref[0, 3:4, :]
    gw_row = gvx2 - gvx1
    gh_row = gvy2 - gvy1
    gcx_row = gvx1 + 0.5 * gw_row
    gcy_row = gvy1 + 0.5 * gh_row
    aidx = aidx_s[...]
    gcx = jnp.take_along_axis(jnp.broadcast_to(gcx_row, (_R, _G)), aidx, axis=1)
    gcy = jnp.take_along_axis(jnp.broadcast_to(gcy_row, (_R, _G)), aidx, axis=1)
    gwp = jnp.take_along_axis(jnp.broadcast_to(gw_row, (_R, _G)), aidx, axis=1)
    ghp = jnp.take_along_axis(jnp.broadcast_to(gh_row, (_R, _G)), aidx, axis=1)

    aw = ax2 - ax1
    ah = ay2 - ay1
    acx = ax1 + 0.5 * aw
    acy = ay1 + 0.5 * ah
    tx = (gcx - acx) / aw
    ty = (gcy - acy) / ah
    tw = jnp.log(gwp / aw)
    th = jnp.log(ghp / ah)
    rows = (_smooth_l1(dt_ref[0] - tx) + _smooth_l1(dt_ref[1] - ty)
            + _smooth_l1(dt_ref[2] - tw) + _smooth_l1(dt_ref[3] - th))
    reg_sum = jnp.sum(jnp.where(sel_pos, rows, 0.0))
    reg = reg_sum / jnp.maximum(npos, 1.0)
    reg = jnp.where(npos > 0.0, reg, 0.0)

    cls_ref[0, 0] = cls_loss
    reg_ref[0, 0] = reg


def kernel(image_shape, anchors, rpn_score, rpn_bboxes_txtytwth, gt_bboxes):
    hw = image_shape.astype(jnp.float32).reshape(1, 2)
    gtt = gt_bboxes.T.reshape(4, _G)
    gtv = gtt.reshape(1, 4, _G)
    allin = jnp.concatenate(
        [anchors, rpn_bboxes_txtytwth, rpn_score], axis=1).T.reshape(
            10, _R, _C)
    cls_out, reg_out = pl.pallas_call(
        _loss_kernel,
        out_shape=[jax.ShapeDtypeStruct((1, 1), jnp.float32)] * 2,
        in_specs=[
            pl.BlockSpec(memory_space=pltpu.SMEM),
            pl.BlockSpec(memory_space=pltpu.SMEM),
            pl.BlockSpec(memory_space=pltpu.VMEM),
            pl.BlockSpec(memory_space=pltpu.VMEM),
        ],
        out_specs=[pl.BlockSpec(memory_space=pltpu.SMEM)] * 2,
        scratch_shapes=[pltpu.VMEM((_R, _C), jnp.float32),
                        pltpu.VMEM((_R, _C), jnp.int32)],
    )(hw, gtt, gtv, allin)
    return (cls_out.reshape(()), reg_out.reshape(()))


# speedup vs baseline: 1.8068x; 1.0025x over previous
import jax
import jax.numpy as jnp
from jax.experimental import pallas as pl
from jax.experimental.pallas import tpu as pltpu

_R = 288
_C = 128
_G = 64
_T = _R // 8


def _smooth_l1(d):
    ad = jnp.abs(d)
    return jnp.where(ad < 1.0, 0.5 * d * d, ad - 0.5)


def _loss_kernel(hw_ref, gt_ref, gtv_ref, all_ref,
                 cls_ref, reg_ref, amax_s, aidx_s):
    at_ref = all_ref.at[0:4]
    dt_ref = all_ref.at[4:8]
    st_ref = all_ref.at[8:10]
    h = hw_ref[0, 0]
    w = hw_ref[0, 1]
    gs = []
    for g in range(_G):
        gx1 = gt_ref[0, g]
        gy1 = gt_ref[1, g]
        gx2 = gt_ref[2, g]
        gy2 = gt_ref[3, g]
        gs.append((gx1, gy1, gx2, gy2, (gx2 - gx1) * (gy2 - gy1)))

    ts = 32
    nt = _R // ts
    sub = jax.lax.broadcasted_iota(jnp.int32, (ts, _C), 0)
    lane = jax.lax.broadcasted_iota(jnp.int32, (ts, _C), 1)
    base_idx = sub * _C + lane

    cmax = [jnp.full((ts, _C), -1.0, jnp.float32) for _ in range(_G)]
    cidx = [jnp.zeros((ts, _C), jnp.int32) for _ in range(_G)]

    for t in range(nt):
        sl = slice(ts * t, ts * t + ts)
        ax1 = at_ref[0, sl, :]
        ay1 = at_ref[1, sl, :]
        ax2 = at_ref[2, sl, :]
        ay2 = at_ref[3, sl, :]
        validt = (ax1 >= 0.0) & (ay1 >= 0.0) & (ax2 <= w) & (ay2 <= h)
        ax1d = jnp.where(validt, ax1, 2048.0)
        ax2d = jnp.where(validt, ax2, 2048.0)
        area_a = (ax2d - ax1d) * (ay2 - ay1)
        idxt = base_idx + t * (ts * _C)
        nch = 2
        amax_c = [jnp.full((ts, _C), -1.0, jnp.float32) for _ in range(nch)]
        aidx_c = [jnp.zeros((ts, _C), jnp.int32) for _ in range(nch)]
        for g in range(_G):
            k = g % nch
            gx1, gy1, gx2, gy2, garea = gs[g]
            ix1 = jnp.maximum(ax1d, gx1)
            iy1 = jnp.maximum(ay1, gy1)
            ix2 = jnp.minimum(ax2d, gx2)
            iy2 = jnp.minimum(ay2, gy2)
            iw = jnp.maximum(ix2 - ix1, 0.0)
            ih = iy2 - iy1
            inter = iw * ih
            union = (area_a + garea) - inter
            iou = inter / union
            upd = iou > amax_c[k]
            amax_c[k] = jnp.maximum(amax_c[k], iou)
            aidx_c[k] = jnp.where(upd, g, aidx_c[k])
            updc = iou > cmax[g]
            cmax[g] = jnp.maximum(cmax[g], iou)
            cidx[g] = jnp.where(updc, idxt, cidx[g])
        amax, aidx = amax_c[0], aidx_c[0]
        for k in range(1, nch):
            take = (amax_c[k] > amax) | ((amax_c[k] == amax)
                                         & (aidx_c[k] < aidx))
            amax = jnp.maximum(amax, amax_c[k])
            aidx = jnp.where(take, aidx_c[k], aidx)
        amax_s[sl, :] = amax
        aidx_s[sl, :] = aidx

    ridx = jax.lax.broadcasted_iota(jnp.int32, (_R, _C), 0)
    cidx_full = jax.lax.broadcasted_iota(jnp.int32, (_R, _C), 1)
    idx_full = ridx * _C + cidx_full
    forced = jnp.zeros((_R, _C), jnp.bool_)
    for g in range(_G):
        maxv = jnp.max(cmax[g])
        cand = jnp.where(cmax[g] == maxv, cidx[g], jnp.int32(2**30))
        gidx = jnp.min(cand)
        forced = forced | (idx_full == gidx)

    ax1 = at_ref[0]
    ay1 = at_ref[1]
    ax2 = at_ref[2]
    ay2 = at_ref[3]
    valid = (ax1 >= 0.0) & (ay1 >= 0.0) & (ax2 <= w) & (ay2 <= h)
    amax = amax_s[...]
    pos = valid & (forced | (amax >= 0.7))
    neg = valid & (amax < 0.3) & jnp.logical_not(forced)

    ic0 = jax.lax.broadcasted_iota(jnp.int32, (_C, _C), 0)
    ic1 = jax.lax.broadcasted_iota(jnp.int32, (_C, _C), 1)
    u_in = (ic0 <= ic1).astype(jnp.float32)
    ir0 = jax.lax.broadcasted_iota(jnp.int32, (_R, _R), 0)
    ir1 = jax.lax.broadcasted_iota(jnp.int32, (_R, _R), 1)
    t_ex = (ir1 < ir0).astype(jnp.float32)

    pos_f = pos.astype(jnp.float32)
    neg_f = neg.astype(jnp.float32)
    pos_rs = jnp.sum(pos_f, axis=1, keepdims=True)
    neg_rs = jnp.sum(neg_f, axis=1, keepdims=True)
    pc = (jnp.dot(t_ex, pos_rs, preferred_element_type=jnp.float32)
          + jnp.dot(pos_f, u_in, preferred_element_type=jnp.float32))
    nc = (jnp.dot(t_ex, neg_rs, preferred_element_type=jnp.float32)
          + jnp.dot(neg_f, u_in, preferred_element_type=jnp.float32))

    npos_total = jnp.sum(pos_f)
    npos = jnp.minimum(npos_total, 128.0)
    sel_pos = pos & (pc <= 128.0)
    nneg_total = jnp.sum(neg_f)
    nneg = jnp.minimum(nneg_total, 256.0 - npos)
    sel_neg = neg & (nc <= 256.0 - npos)

    s0 = st_ref[0]
    s1 = st_ref[1]
    mx = jnp.maximum(s0, s1)
    lse = mx + jnp.log(jnp.exp(s0 - mx) + jnp.exp(s1 - mx))
    ce = jnp.where(sel_pos, lse - s1, 0.0) + jnp.where(sel_neg, lse - s0, 0.0)
    cls_loss = 3.0 * jnp.sum(ce) / (npos + nneg)

    gvx1 = gtv_ref[0, 0:1, :]
    gvy1 = gtv_ref[0, 1:2, :]
    gvx2 = gtv_ref[0, 2:3, :]
    gvy2 = gtv_ref[0, 3:4, :]
    gw_row = gvx2 - gvx1
    gh_row = gvy2 - gvy1
    gcx_row = gvx1 + 0.5 * gw_row
    gcy_row = gvy1 + 0.5 * gh_row
    aidx = aidx_s[...]
    gcx = jnp.take_along_axis(jnp.broadcast_to(gcx_row, (_R, _G)), aidx, axis=1)
    gcy = jnp.take_along_axis(jnp.broadcast_to(gcy_row, (_R, _G)), aidx, axis=1)
    gwp = jnp.take_along_axis(jnp.broadcast_to(gw_row, (_R, _G)), aidx, axis=1)
    ghp = jnp.take_along_axis(jnp.broadcast_to(gh_row, (_R, _G)), aidx, axis=1)

    aw = ax2 - ax1
    ah = ay2 - ay1
    acx = ax1 + 0.5 * aw
    acy = ay1 + 0.5 * ah
    tx = (gcx - acx) / aw
    ty = (gcy - acy) / ah
    tw = jnp.log(gwp / aw)
    th = jnp.log(ghp / ah)
    rows = (_smooth_l1(dt_ref[0] - tx) + _smooth_l1(dt_ref[1] - ty)
            + _smooth_l1(dt_ref[2] - tw) + _smooth_l1(dt_ref[3] - th))
    reg_sum = jnp.sum(jnp.where(sel_pos, rows, 0.0))
    reg = reg_sum / jnp.maximum(npos, 1.0)
    reg = jnp.where(npos > 0.0, reg, 0.0)

    cls_ref[0, 0] = cls_loss
    reg_ref[0, 0] = reg


def kernel(image_shape, anchors, rpn_score, rpn_bboxes_txtytwth, gt_bboxes):
    hw = image_shape.astype(jnp.float32).reshape(1, 2)
    gtt = gt_bboxes.T.reshape(4, _G)
    gtv = gtt.reshape(1, 4, _G)
    allin = jnp.concatenate(
        [anchors, rpn_bboxes_txtytwth, rpn_score], axis=1).T.reshape(
            10, _R, _C)
    cls_out, reg_out = pl.pallas_call(
        _loss_kernel,
        out_shape=[jax.ShapeDtypeStruct((1, 1), jnp.float32)] * 2,
        in_specs=[
            pl.BlockSpec(memory_space=pltpu.SMEM),
            pl.BlockSpec(memory_space=pltpu.SMEM),
            pl.BlockSpec(memory_space=pltpu.VMEM),
            pl.BlockSpec(memory_space=pltpu.VMEM),
        ],
        out_specs=[pl.BlockSpec(memory_space=pltpu.SMEM)] * 2,
        scratch_shapes=[pltpu.VMEM((_R, _C), jnp.float32),
                        pltpu.VMEM((_R, _C), jnp.int32)],
    )(hw, gtt, gtv, allin)
    return (cls_out.reshape(()), reg_out.reshape(()))
